# baseline (speedup 1.0000x reference)
import functools

import jax
import jax.numpy as jnp
import numpy as np
from jax import lax
from jax.experimental import pallas as pl
from jax.experimental.pallas import tpu as pltpu

_K0 = np.uint32(0)
_K1 = np.uint32(42)
_KS2 = np.uint32(_K0 ^ _K1 ^ np.uint32(0x1BD11BDA))
_ROT = (13, 15, 26, 6, 17, 29, 16, 24, 13, 15, 26, 6, 17, 29, 16, 24, 13, 15, 26, 6)
_INJ = ((1, 2, 1), (2, 0, 2), (0, 1, 3), (1, 2, 4), (2, 0, 5))
_TINY = np.float32(np.finfo(np.float32).tiny)


def _threefry_bits(linear_idx_u32):
    ks = (_K0, _K1, _KS2)
    x0 = jnp.full(linear_idx_u32.shape, _K0, dtype=jnp.uint32)
    x1 = linear_idx_u32 + _K1
    for chunk, (a, b, c) in enumerate(_INJ):
        for r in _ROT[4 * chunk:4 * chunk + 4]:
            x0 = x0 + x1
            x1 = (x1 << np.uint32(r)) | (x1 >> np.uint32(32 - r))
            x1 = x1 ^ x0
        x0 = x0 + ks[a]
        x1 = x1 + ks[b] + np.uint32(c)
    return x0 ^ x1


_INT_MAX = np.int32(np.iinfo(np.int32).max)
_NEG_INF = np.float32(-np.inf)


def _row_kernel(x_ref, probs_ref, samp_ref, *, ncols, nsub, cw, clane, nlc, ntail):
    row = pl.program_id(0)

    m = jnp.max(x_ref[...])

    def step(off, cl, carry):
        s, bv, bi = carry
        xc = x_ref[:, :, pl.ds(off, cl)]
        e = jnp.exp(xc - m)
        probs_ref[:, :, pl.ds(off, cl)] = e
        s = s + jnp.sum(e)

        subl = lax.broadcasted_iota(jnp.int32, (1, nsub, cl), 1) * cw
        lane = lax.broadcasted_iota(jnp.int32, (1, nsub, cl), 2)
        icol = subl + lane + off

        idx = icol.astype(jnp.uint32) + jnp.uint32(row * ncols)
        bits = _threefry_bits(idx)
        fb = (bits >> np.uint32(9)) | np.uint32(0x3F800000)
        f = lax.bitcast_convert_type(fb, jnp.float32) - np.float32(1.0)
        u = jnp.maximum(_TINY, f * np.float32(1.0 - _TINY) + _TINY)
        g = -jnp.log(-jnp.log(u))

        v = xc + g
        vm = jnp.max(v)
        vi = jnp.min(jnp.where(v == vm, icol, _INT_MAX))
        upd = (vm > bv) | ((vm == bv) & (vi < bi))
        bv = jnp.where(upd, vm, bv)
        bi = jnp.where(upd, vi, bi)
        return s, bv, bi

    carry = (jnp.float32(0.0), _NEG_INF, _INT_MAX)
    if nlc:
        carry = lax.fori_loop(
            0, nlc, lambda k, c: step(k * clane, clane, c), carry)
    if ntail:
        carry = step(nlc * clane, ntail, carry)
    s, _, bi = carry

    probs_ref[...] = probs_ref[...] * (np.float32(1.0) / s)
    samp_ref[...] = jnp.full(samp_ref.shape, bi, dtype=jnp.int32)


def kernel(logits):
    nrows, ncols = logits.shape
    nsub = 8 if ncols % 8 == 0 else 1
    cw = ncols // nsub
    nfull = cw // 128
    clane = min(1024, max(nfull, 1) * 128)
    nlc = (nfull * 128) // clane
    ntail = cw - nlc * clane
    logits3 = logits.reshape(nrows, nsub, cw)
    nchunks = nsub
    probs3, samples3 = pl.pallas_call(
        functools.partial(_row_kernel, ncols=ncols, nsub=nsub, cw=cw,
                          clane=clane, nlc=nlc, ntail=ntail),
        grid=(nrows,),
        in_specs=[pl.BlockSpec((1, nchunks, cw), lambda r: (r, 0, 0))],
        out_specs=[
            pl.BlockSpec((1, nchunks, cw), lambda r: (r, 0, 0)),
            pl.BlockSpec((1, 1, 128), lambda r: (r, 0, 0)),
        ],
        out_shape=[
            jax.ShapeDtypeStruct((nrows, nchunks, cw), jnp.float32),
            jax.ShapeDtypeStruct((nrows, 1, 128), jnp.int32),
        ],
        compiler_params=pltpu.CompilerParams(
            dimension_semantics=("arbitrary",),
        ),
    )(logits3)
    samples = samples3[:, 0, 0]
    probs = probs3.reshape(nrows, ncols)
    return (samples, probs)


# speedup vs baseline: 1.8178x; 1.8178x over previous
import functools

import jax
import jax.numpy as jnp
import numpy as np
from jax import lax
from jax.experimental import pallas as pl
from jax.experimental.pallas import tpu as pltpu

_K0 = np.uint32(0)
_K1 = np.uint32(42)
_KS2 = np.uint32(_K0 ^ _K1 ^ np.uint32(0x1BD11BDA))
_ROT = (13, 15, 26, 6, 17, 29, 16, 24, 13, 15, 26, 6, 17, 29, 16, 24, 13, 15, 26, 6)
_INJ = ((1, 2, 1), (2, 0, 2), (0, 1, 3), (1, 2, 4), (2, 0, 5))
_TINY = np.float32(np.finfo(np.float32).tiny)


def _threefry_bits(linear_idx_u32):
    ks = (_K0, _K1, _KS2)
    x0 = jnp.full(linear_idx_u32.shape, _K0, dtype=jnp.uint32)
    x1 = linear_idx_u32 + _K1
    for chunk, (a, b, c) in enumerate(_INJ):
        for r in _ROT[4 * chunk:4 * chunk + 4]:
            x0 = x0 + x1
            x1 = (x1 << np.uint32(r)) | (x1 >> np.uint32(32 - r))
            x1 = x1 ^ x0
        x0 = x0 + ks[a]
        x1 = x1 + ks[b] + np.uint32(c)
    return x0 ^ x1


_INT_MAX = np.int32(np.iinfo(np.int32).max)
_NEG_INF = np.float32(-np.inf)


def _row_kernel(x_ref, probs_ref, samp_ref, *, ncols, nsub, cw, clane, nlc, ntail):
    row = pl.program_id(0)

    m = jnp.max(x_ref[...])

    def step(off, cl, carry):
        s, bv, bi = carry
        xc = x_ref[:, :, pl.ds(off, cl)]
        e = jnp.exp(xc - m)
        probs_ref[:, :, pl.ds(off, cl)] = e
        s = s + jnp.sum(e)

        subl = lax.broadcasted_iota(jnp.int32, (1, nsub, cl), 1) * cw
        lane = lax.broadcasted_iota(jnp.int32, (1, nsub, cl), 2)
        icol = subl + lane + off

        idx = icol.astype(jnp.uint32) + jnp.uint32(row * ncols)
        bits = _threefry_bits(idx)
        fb = (bits >> np.uint32(9)) | np.uint32(0x3F800000)
        f = lax.bitcast_convert_type(fb, jnp.float32) - np.float32(1.0)
        u = jnp.maximum(_TINY, f * np.float32(1.0 - _TINY) + _TINY)
        g = -jnp.log(-jnp.log(u))

        v = xc + g
        vm = jnp.max(v)
        vi = jnp.min(jnp.where(v == vm, icol, _INT_MAX))
        upd = (vm > bv) | ((vm == bv) & (vi < bi))
        bv = jnp.where(upd, vm, bv)
        bi = jnp.where(upd, vi, bi)
        return s, bv, bi

    carry = (jnp.float32(0.0), _NEG_INF, _INT_MAX)
    for k in range(nlc):
        carry = step(k * clane, clane, carry)
    if ntail:
        carry = step(nlc * clane, ntail, carry)
    s, _, bi = carry

    probs_ref[...] = probs_ref[...] * (np.float32(1.0) / s)
    samp_ref[...] = jnp.full(samp_ref.shape, bi, dtype=jnp.int32)


def kernel(logits):
    nrows, ncols = logits.shape
    nsub = 8 if ncols % 8 == 0 else 1
    cw = ncols // nsub
    nfull = cw // 128
    clane = min(1024, max(nfull, 1) * 128)
    nlc = (nfull * 128) // clane
    ntail = cw - nlc * clane
    logits3 = logits.reshape(nrows, nsub, cw)
    nchunks = nsub
    probs3, samples3 = pl.pallas_call(
        functools.partial(_row_kernel, ncols=ncols, nsub=nsub, cw=cw,
                          clane=clane, nlc=nlc, ntail=ntail),
        grid=(nrows,),
        in_specs=[pl.BlockSpec((1, nchunks, cw), lambda r: (r, 0, 0))],
        out_specs=[
            pl.BlockSpec((1, nchunks, cw), lambda r: (r, 0, 0)),
            pl.BlockSpec((1, 1, 128), lambda r: (r, 0, 0)),
        ],
        out_shape=[
            jax.ShapeDtypeStruct((nrows, nchunks, cw), jnp.float32),
            jax.ShapeDtypeStruct((nrows, 1, 128), jnp.int32),
        ],
        compiler_params=pltpu.CompilerParams(
            dimension_semantics=("arbitrary",),
        ),
    )(logits3)
    samples = samples3[:, 0, 0]
    probs = probs3.reshape(nrows, ncols)
    return (samples, probs)
